# baseline (speedup 1.0000x reference)
import jax
import jax.numpy as jnp
from jax import lax
from jax.experimental import pallas as pl
from jax.experimental.pallas import tpu as pltpu

BATCH = 4096
EMB = 4096
DIM = 8192
CH = 512
NCH = BATCH // CH
NB = 4


def _tc_body(idx_ref, dv_ref, x_any, w_ref, b_ref, cache_ref, out_any,
             *scratch):
    bufs = scratch[:NB]
    in_sems = scratch[NB:2 * NB]
    out_sems = scratch[2 * NB:3 * NB]

    idx = idx_ref[0]
    iota = lax.broadcasted_iota(jnp.int32, (1, DIM), 1)
    sel = (iota == idx).astype(jnp.float32)
    w = jnp.sum(w_ref[...] * sel)
    b = jnp.sum(b_ref[...] * sel)
    dv = jnp.clip(dv_ref[0], 0.9, 1.0)
    addend = dv * cache_ref[...] + b

    def start_in(g):
        return pltpu.make_async_copy(
            x_any.at[pl.ds(g * CH, CH), :], bufs[g % NB], in_sems[g % NB])

    def start_out(g):
        return pltpu.make_async_copy(
            bufs[g % NB], out_any.at[pl.ds(g * CH, CH), :], out_sems[g % NB])

    for g in range(min(NB - 1, NCH)):
        start_in(g).start()
    outs = {}
    for g in range(NCH):
        start_in(g).wait()
        buf = bufs[g % NB]
        buf[...] = buf[...] * w + addend
        outs[g] = start_out(g)
        outs[g].start()
        if g + NB - 1 < NCH:
            if g >= 1:
                outs[g - 1].wait()
            start_in(g + NB - 1).start()
    for g in range(max(0, NCH - NB), NCH):
        outs[g].wait()


_call = pl.pallas_call(
    _tc_body,
    in_specs=[
        pl.BlockSpec(memory_space=pltpu.SMEM),
        pl.BlockSpec(memory_space=pltpu.SMEM),
        pl.BlockSpec(memory_space=pl.ANY),
        pl.BlockSpec(memory_space=pltpu.VMEM),
        pl.BlockSpec(memory_space=pltpu.VMEM),
        pl.BlockSpec(memory_space=pltpu.VMEM),
    ],
    out_specs=pl.BlockSpec(memory_space=pl.ANY),
    out_shape=jax.ShapeDtypeStruct((BATCH, EMB), jnp.float32),
    scratch_shapes=(
        [pltpu.VMEM((CH, EMB), jnp.float32)] * NB
        + [pltpu.SemaphoreType.DMA] * (2 * NB)
    ),
)


@jax.jit
def kernel(x, index, weight, bias, decay_value, cache):
    idx1 = jnp.asarray(index, jnp.int32).reshape(1)
    dv1 = decay_value.astype(jnp.float32).reshape(1)
    return _call(idx1, dv1, x, weight.reshape(1, DIM),
                 bias.reshape(1, DIM), cache.reshape(1, EMB))


# speedup vs baseline: 1.0141x; 1.0141x over previous
import jax
import jax.numpy as jnp
from jax import lax
from jax.experimental import pallas as pl
from jax.experimental.pallas import tpu as pltpu

BATCH = 4096
EMB = 4096
DIM = 8192
CH = 512
NCH = BATCH // CH
NB = 4


class _Pair:
    def __init__(self, a, b):
        self._a, self._b = a, b

    def start(self):
        self._a.start()
        self._b.start()

    def wait(self):
        self._a.wait()
        self._b.wait()


def _tc_body(idx_ref, dv_ref, x_any, w_ref, b_ref, cache_ref, out_any,
             *scratch):
    bufs = scratch[:NB]
    in_sems = scratch[NB:2 * NB]
    out_sems = scratch[2 * NB:3 * NB]
    in_sems2 = scratch[3 * NB:4 * NB]
    out_sems2 = scratch[4 * NB:5 * NB]

    idx = idx_ref[0]
    iota = lax.broadcasted_iota(jnp.int32, (1, DIM), 1)
    sel = (iota == idx).astype(jnp.float32)
    w = jnp.sum(w_ref[...] * sel)
    b = jnp.sum(b_ref[...] * sel)
    dv = jnp.clip(dv_ref[0], 0.9, 1.0)
    addend = dv * cache_ref[...] + b

    H = CH // 2

    def start_in(g):
        c1 = pltpu.make_async_copy(
            x_any.at[pl.ds(g * CH, H), :], bufs[g % NB].at[pl.ds(0, H), :],
            in_sems[g % NB])
        c2 = pltpu.make_async_copy(
            x_any.at[pl.ds(g * CH + H, H), :], bufs[g % NB].at[pl.ds(H, H), :],
            in_sems2[g % NB])
        return _Pair(c1, c2)

    def start_out(g):
        c1 = pltpu.make_async_copy(
            bufs[g % NB].at[pl.ds(0, H), :], out_any.at[pl.ds(g * CH, H), :],
            out_sems[g % NB])
        c2 = pltpu.make_async_copy(
            bufs[g % NB].at[pl.ds(H, H), :], out_any.at[pl.ds(g * CH + H, H), :],
            out_sems2[g % NB])
        return _Pair(c1, c2)

    for g in range(min(NB - 1, NCH)):
        start_in(g).start()
    outs = {}
    for g in range(NCH):
        start_in(g).wait()
        buf = bufs[g % NB]
        buf[...] = buf[...] * w + addend
        outs[g] = start_out(g)
        outs[g].start()
        if g + NB - 1 < NCH:
            if g >= 1:
                outs[g - 1].wait()
            start_in(g + NB - 1).start()
    for g in range(max(0, NCH - NB), NCH):
        outs[g].wait()


_call = pl.pallas_call(
    _tc_body,
    in_specs=[
        pl.BlockSpec(memory_space=pltpu.SMEM),
        pl.BlockSpec(memory_space=pltpu.SMEM),
        pl.BlockSpec(memory_space=pl.ANY),
        pl.BlockSpec(memory_space=pltpu.VMEM),
        pl.BlockSpec(memory_space=pltpu.VMEM),
        pl.BlockSpec(memory_space=pltpu.VMEM),
    ],
    out_specs=pl.BlockSpec(memory_space=pl.ANY),
    out_shape=jax.ShapeDtypeStruct((BATCH, EMB), jnp.float32),
    scratch_shapes=(
        [pltpu.VMEM((CH, EMB), jnp.float32)] * NB
        + [pltpu.SemaphoreType.DMA] * (4 * NB)
    ),
)


@jax.jit
def kernel(x, index, weight, bias, decay_value, cache):
    idx1 = jnp.asarray(index, jnp.int32).reshape(1)
    dv1 = decay_value.astype(jnp.float32).reshape(1)
    return _call(idx1, dv1, x, weight.reshape(1, DIM),
                 bias.reshape(1, DIM), cache.reshape(1, EMB))
